# baseline (speedup 1.0000x reference)
import jax
import jax.numpy as jnp
from jax.experimental import pallas as pl
from jax.experimental.pallas import tpu as pltpu

_OFF_HALF = 9
_N_SEG = 4
_TH = 128
_RC = 16
_CC = 128


def _loss_kernel(off_ref, flow_ref, out_ref, off_bf, flow_bf):
    jt = pl.program_id(1)
    w = off_ref.shape[3]

    off_bf[...] = off_ref[0].astype(jnp.bfloat16)
    flow_bf[...] = flow_ref[0].astype(jnp.bfloat16)

    acc = None
    for r in range(0, _TH, _RC):
        for c in range(0, w, _CC):
            rs = slice(r, r + _RC)
            cs = slice(c, c + _CC)
            seg = []
            for j in range(_N_SEG):
                u = flow_bf[j, rs, cs]
                v = flow_bf[j + 1, rs, cs]
                uu = u * u + v * v
                inv = 1.0 / uu
                lo = jnp.minimum(0.0, u) * uu
                hi = jnp.maximum(0.0, u) * uu
                lo = jnp.where(uu > 0.0, lo, 1.0)
                hi = jnp.where(uu > 0.0, hi, 0.0)
                seg.append((u, v, inv, lo, hi))
            msum = None
            for i in range(_OFF_HALF):
                x = off_bf[i, rs, cs]
                y = off_bf[_OFF_HALF + i, rs, cs]
                xx = x * x
                d1sq = xx + y * y
                msq = None
                for (u, v, inv, lo, hi) in seg:
                    s = u * (xx + v * y)
                    inside = (lo <= s) & (s <= hi)
                    t = v * x - u * y
                    perpsq = t * t * inv
                    dx = x - u
                    dy = y - v
                    d2sq = dx * dx + dy * dy
                    md = jnp.where(inside, perpsq, jnp.minimum(d1sq, d2sq))
                    msq = md if msq is None else jnp.minimum(msq, md)
                m = jnp.sqrt(msq)
                msum = m if msum is None else msum + m
            msum32 = msum.astype(jnp.float32)
            acc = msum32 if acc is None else acc + msum32

    s81 = jnp.sum(acc, axis=-1, keepdims=True)
    s11 = jnp.sum(s81, axis=0, keepdims=True)
    part = jnp.broadcast_to(s11, (8, 128))

    @pl.when(jt == 0)
    def _():
        out_ref[...] = jnp.zeros_like(out_ref)

    out_ref[...] += part


def kernel(offset, optical_flow):
    b, c_off, h, w = offset.shape
    of_num = optical_flow.shape[1] // 2
    flow = optical_flow[:, :of_num + 1]
    ht = h // _TH

    out = pl.pallas_call(
        _loss_kernel,
        out_shape=jax.ShapeDtypeStruct((b * 8, 128), jnp.float32),
        grid=(b, ht),
        in_specs=[
            pl.BlockSpec((1, c_off, _TH, w), lambda i, j: (i, 0, j, 0)),
            pl.BlockSpec((1, of_num + 1, _TH, w), lambda i, j: (i, 0, j, 0)),
        ],
        out_specs=pl.BlockSpec((8, 128), lambda i, j: (i, 0)),
        scratch_shapes=[
            pltpu.VMEM((c_off, _TH, w), jnp.bfloat16),
            pltpu.VMEM((of_num + 1, _TH, w), jnp.bfloat16),
        ],
        compiler_params=pltpu.CompilerParams(
            dimension_semantics=("parallel", "arbitrary"),
        ),
        name="position_loss_val",
    )(offset, flow)

    total = jnp.sum(out[::8, 0])
    return total / (_OFF_HALF * h * w)


# speedup vs baseline: 1.1394x; 1.1394x over previous
import jax
import jax.numpy as jnp
from jax.experimental import pallas as pl
from jax.experimental.pallas import tpu as pltpu

_OFF_HALF = 9
_N_SEG = 4
_TH = 128
_RC = 16
_CC = 128


def _loss_kernel(off_ref, flow_ref, out_ref, off_bf, flow_bf):
    jt = pl.program_id(1)
    w = off_ref.shape[3]

    off_bf[...] = off_ref[0].astype(jnp.bfloat16)
    flow_bf[...] = flow_ref[0].astype(jnp.bfloat16)

    acc = None
    for r in range(0, _TH, _RC):
        for c in range(0, w, _CC):
            rs = slice(r, r + _RC)
            cs = slice(c, c + _CC)
            seg = []
            for j in range(_N_SEG):
                u = flow_bf[j, rs, cs]
                v = flow_bf[j + 1, rs, cs]
                uu = u * u + v * v
                inv = 1.0 / uu
                lo = jnp.minimum(0.0, u) * uu
                hi = jnp.maximum(0.0, u) * uu
                lo = jnp.where(uu > 0.0, lo, 1.0)
                hi = jnp.where(uu > 0.0, hi, 0.0)
                seg.append((u, v, inv, lo, hi))
            msum = None
            for i in range(_OFF_HALF):
                x = off_bf[i, rs, cs]
                y = off_bf[_OFF_HALF + i, rs, cs]
                xx = x * x
                d1sq = xx + y * y
                msq = None
                for (u, v, inv, lo, hi) in seg:
                    s = u * (xx + v * y)
                    inside = (lo <= s) & (s <= hi)
                    t = v * x - u * y
                    perpsq = t * t * inv
                    dx = x - u
                    dy = y - v
                    d2sq = dx * dx + dy * dy
                    md = jnp.where(inside, perpsq, jnp.minimum(d1sq, d2sq))
                    msq = md if msq is None else jnp.minimum(msq, md)
                m = jnp.sqrt(msq)
                msum = m if msum is None else msum + m
            msum32 = msum.astype(jnp.float32)
            acc = msum32 if acc is None else acc + msum32

    s81 = jnp.sum(acc, axis=-1, keepdims=True)
    s11 = jnp.sum(s81, axis=0, keepdims=True)
    part = jnp.broadcast_to(s11, (8, 128))

    @pl.when(jt == 0)
    def _():
        out_ref[...] = jnp.zeros_like(out_ref)

    out_ref[...] += part


def kernel(offset, optical_flow):
    b, c_off, h, w = offset.shape
    of_num = optical_flow.shape[1] // 2
    ht = h // _TH

    out = pl.pallas_call(
        _loss_kernel,
        out_shape=jax.ShapeDtypeStruct((b * 8, 128), jnp.float32),
        grid=(b, ht),
        in_specs=[
            pl.BlockSpec((1, c_off, _TH, w), lambda i, j: (i, 0, j, 0)),
            pl.BlockSpec((1, of_num + 1, _TH, w), lambda i, j: (i, 0, j, 0)),
        ],
        out_specs=pl.BlockSpec((8, 128), lambda i, j: (i, 0)),
        scratch_shapes=[
            pltpu.VMEM((c_off, _TH, w), jnp.bfloat16),
            pltpu.VMEM((of_num + 1, _TH, w), jnp.bfloat16),
        ],
        compiler_params=pltpu.CompilerParams(
            dimension_semantics=("parallel", "arbitrary"),
        ),
        name="position_loss_val",
    )(offset, optical_flow)

    total = jnp.sum(out[::8, 0])
    return total / (_OFF_HALF * h * w)


# speedup vs baseline: 1.2082x; 1.0604x over previous
import jax
import jax.numpy as jnp
from jax.experimental import pallas as pl
from jax.experimental.pallas import tpu as pltpu

_OFF_HALF = 9
_N_SEG = 4
_TH = 128
_RC = 16
_CC = 128


def _loss_kernel(off_ref, flow_ref, out_ref, off_bf, flow_bf):
    jt = pl.program_id(1)
    w = off_ref.shape[3]

    off_bf[...] = off_ref[0].astype(jnp.bfloat16)
    flow_bf[...] = flow_ref[0].astype(jnp.bfloat16)

    acc = None
    for r in range(0, _TH, _RC):
        for c in range(0, w, _CC):
            rs = slice(r, r + _RC)
            cs = slice(c, c + _CC)
            seg = []
            for j in range(_N_SEG):
                u = flow_bf[j, rs, cs]
                v = flow_bf[j + 1, rs, cs]
                uu = u * u + v * v
                inv = 1.0 / uu
                lo = jnp.minimum(0.0, u) * uu
                hi = jnp.maximum(0.0, u) * uu
                lo = jnp.where(uu > 0.0, lo, 1.0)
                hi = jnp.where(uu > 0.0, hi, 0.0)
                seg.append((u, v, inv, lo, hi))
            msum = None
            for i in range(_OFF_HALF):
                x = off_bf[i, rs, cs]
                y = off_bf[_OFF_HALF + i, rs, cs]
                xx = x * x
                d1sq = xx + y * y
                msq = None
                any_out = None
                for (u, v, inv, lo, hi) in seg:
                    s = u * (xx + v * y)
                    out = (s < lo) | (s > hi)
                    t = v * x - u * y
                    perpsq = t * t * inv
                    dx = x - u
                    dy = y - v
                    d2sq = dx * dx + dy * dy
                    md = jnp.where(out, d2sq, perpsq)
                    msq = md if msq is None else jnp.minimum(msq, md)
                    any_out = out if any_out is None else any_out | out
                msq = jnp.minimum(msq, jnp.where(any_out, d1sq, jnp.inf))
                m = msq * jax.lax.rsqrt(jnp.maximum(msq, 1e-30))
                msum = m if msum is None else msum + m
            msum32 = msum.astype(jnp.float32)
            acc = msum32 if acc is None else acc + msum32

    s81 = jnp.sum(acc, axis=-1, keepdims=True)
    s11 = jnp.sum(s81, axis=0, keepdims=True)
    part = jnp.broadcast_to(s11, (8, 128))

    @pl.when(jt == 0)
    def _():
        out_ref[...] = jnp.zeros_like(out_ref)

    out_ref[...] += part


def kernel(offset, optical_flow):
    b, c_off, h, w = offset.shape
    of_num = optical_flow.shape[1] // 2
    ht = h // _TH

    out = pl.pallas_call(
        _loss_kernel,
        out_shape=jax.ShapeDtypeStruct((b * 8, 128), jnp.float32),
        grid=(b, ht),
        in_specs=[
            pl.BlockSpec((1, c_off, _TH, w), lambda i, j: (i, 0, j, 0)),
            pl.BlockSpec((1, of_num + 1, _TH, w), lambda i, j: (i, 0, j, 0)),
        ],
        out_specs=pl.BlockSpec((8, 128), lambda i, j: (i, 0)),
        scratch_shapes=[
            pltpu.VMEM((c_off, _TH, w), jnp.bfloat16),
            pltpu.VMEM((of_num + 1, _TH, w), jnp.bfloat16),
        ],
        compiler_params=pltpu.CompilerParams(
            dimension_semantics=("parallel", "arbitrary"),
        ),
        name="position_loss_val",
    )(offset, optical_flow)

    total = jnp.sum(out[::8, 0])
    return total / (_OFF_HALF * h * w)


# speedup vs baseline: 1.2198x; 1.0096x over previous
import jax
import jax.numpy as jnp
from jax.experimental import pallas as pl
from jax.experimental.pallas import tpu as pltpu

_OFF_HALF = 9
_N_SEG = 4
_TH = 128
_RC = 16
_CC = 128


def _loss_kernel(off_ref, flow_ref, out_ref, off_bf, flow_bf):
    jt = pl.program_id(1)
    w = off_ref.shape[3]

    off_bf[...] = off_ref[0].astype(jnp.bfloat16)
    flow_bf[...] = flow_ref[0].astype(jnp.bfloat16)

    acc = None
    for r in range(0, _TH, _RC):
        for c in range(0, w, _CC):
            rs = slice(r, r + _RC)
            cs = slice(c, c + _CC)
            seg = []
            for j in range(_N_SEG):
                u = flow_bf[j, rs, cs]
                v = flow_bf[j + 1, rs, cs]
                uu = u * u + v * v
                inv = 1.0 / uu
                wj = u * uu
                lo = jnp.minimum(0.0, wj)
                hi = jnp.maximum(0.0, wj)
                hi = jnp.where(uu > 0.0, hi, -1.0)
                seg.append((u, v, inv, lo, hi))
            msum = None
            for i in range(_OFF_HALF):
                x = off_bf[i, rs, cs]
                y = off_bf[_OFF_HALF + i, rs, cs]
                xx = x * x
                d1sq = xx + y * y
                msq = None
                any_out = None
                for (u, v, inv, lo, hi) in seg:
                    s = u * (xx + v * y)
                    out = (s < lo) | (s > hi)
                    t = v * x - u * y
                    perpsq = t * t * inv
                    dx = x - u
                    dy = y - v
                    d2sq = dx * dx + dy * dy
                    md = jnp.where(out, d2sq, perpsq)
                    msq = md if msq is None else jnp.minimum(msq, md)
                    any_out = out if any_out is None else any_out | out
                msq = jnp.minimum(msq, jnp.where(any_out, d1sq, jnp.inf))
                m = msq * jax.lax.rsqrt(jnp.maximum(msq, 1e-30))
                msum = m if msum is None else msum + m
            msum32 = msum.astype(jnp.float32)
            acc = msum32 if acc is None else acc + msum32

    s81 = jnp.sum(acc, axis=-1, keepdims=True)
    s11 = jnp.sum(s81, axis=0, keepdims=True)
    part = jnp.broadcast_to(s11, (8, 128))

    @pl.when((pl.program_id(0) == 0) & (jt == 0))
    def _():
        out_ref[...] = jnp.zeros_like(out_ref)

    out_ref[...] += part


def kernel(offset, optical_flow):
    b, c_off, h, w = offset.shape
    of_num = optical_flow.shape[1] // 2
    ht = h // _TH

    out = pl.pallas_call(
        _loss_kernel,
        out_shape=jax.ShapeDtypeStruct((8, 128), jnp.float32),
        grid=(b, ht),
        in_specs=[
            pl.BlockSpec((1, c_off, _TH, w), lambda i, j: (i, 0, j, 0)),
            pl.BlockSpec((1, of_num + 1, _TH, w), lambda i, j: (i, 0, j, 0)),
        ],
        out_specs=pl.BlockSpec((8, 128), lambda i, j: (0, 0)),
        scratch_shapes=[
            pltpu.VMEM((c_off, _TH, w), jnp.bfloat16),
            pltpu.VMEM((of_num + 1, _TH, w), jnp.bfloat16),
        ],
        compiler_params=pltpu.CompilerParams(
            dimension_semantics=("arbitrary", "arbitrary"),
        ),
        name="position_loss_val",
    )(offset, optical_flow)

    return out[0, 0] / (_OFF_HALF * h * w)


# speedup vs baseline: 1.2209x; 1.0009x over previous
import jax
import jax.numpy as jnp
from jax.experimental import pallas as pl
from jax.experimental.pallas import tpu as pltpu

_OFF_HALF = 9
_N_SEG = 4
_TH = 128
_RC = 16
_CC = 128


def _loss_kernel(off_ref, flow_ref, out_ref, off_bf, flow_bf):
    jt = pl.program_id(1)
    w = off_ref.shape[3]

    off_bf[...] = off_ref[0].astype(jnp.bfloat16)
    flow_bf[...] = flow_ref[0].astype(jnp.bfloat16)

    acc = None
    for r in range(0, _TH, _RC):
        for c in range(0, w, _CC):
            rs = slice(r, r + _RC)
            cs = slice(c, c + _CC)
            ch = [flow_bf[k, rs, cs] for k in range(_N_SEG + 1)]
            seg = []
            for j in range(_N_SEG):
                u = ch[j]
                v = ch[j + 1]
                uu = u * u + v * v
                inv = 1.0 / uu
                wj = u * uu
                lo = jnp.minimum(0.0, wj)
                hi = jnp.maximum(0.0, wj)
                hi = jnp.where(uu > 0.0, hi, -1.0)
                seg.append((u, v, inv, lo, hi))
            msum = None
            for i in range(_OFF_HALF):
                x = off_bf[i, rs, cs]
                y = off_bf[_OFF_HALF + i, rs, cs]
                xx = x * x
                d1sq = xx + y * y
                cy = [ch[k] * y for k in range(_N_SEG + 1)]
                cx = [ch[k] * x for k in range(1, _N_SEG + 1)]
                msq = None
                any_out = None
                for j, (u, v, inv, lo, hi) in enumerate(seg):
                    s = u * (xx + cy[j + 1])
                    out = (s < lo) | (s > hi)
                    t = cx[j] - cy[j]
                    perpsq = t * t * inv
                    dx = x - u
                    dy = y - v
                    d2sq = dx * dx + dy * dy
                    md = jnp.where(out, d2sq, perpsq)
                    msq = md if msq is None else jnp.minimum(msq, md)
                    any_out = out if any_out is None else any_out | out
                msq = jnp.minimum(msq, jnp.where(any_out, d1sq, jnp.inf))
                m = msq * jax.lax.rsqrt(jnp.maximum(msq, 1e-30))
                msum = m if msum is None else msum + m
            msum32 = msum.astype(jnp.float32)
            acc = msum32 if acc is None else acc + msum32

    s81 = jnp.sum(acc, axis=-1, keepdims=True)
    s11 = jnp.sum(s81, axis=0, keepdims=True)
    part = jnp.broadcast_to(s11, (8, 128))

    @pl.when((pl.program_id(0) == 0) & (jt == 0))
    def _():
        out_ref[...] = jnp.zeros_like(out_ref)

    out_ref[...] += part


def kernel(offset, optical_flow):
    b, c_off, h, w = offset.shape
    of_num = optical_flow.shape[1] // 2
    ht = h // _TH

    out = pl.pallas_call(
        _loss_kernel,
        out_shape=jax.ShapeDtypeStruct((8, 128), jnp.float32),
        grid=(b, ht),
        in_specs=[
            pl.BlockSpec((1, c_off, _TH, w), lambda i, j: (i, 0, j, 0)),
            pl.BlockSpec((1, of_num + 1, _TH, w), lambda i, j: (i, 0, j, 0)),
        ],
        out_specs=pl.BlockSpec((8, 128), lambda i, j: (0, 0)),
        scratch_shapes=[
            pltpu.VMEM((c_off, _TH, w), jnp.bfloat16),
            pltpu.VMEM((of_num + 1, _TH, w), jnp.bfloat16),
        ],
        compiler_params=pltpu.CompilerParams(
            dimension_semantics=("arbitrary", "arbitrary"),
        ),
        name="position_loss_val",
    )(offset, optical_flow)

    return out[0, 0] / (_OFF_HALF * h * w)


# speedup vs baseline: 1.2866x; 1.0538x over previous
import jax
import jax.numpy as jnp
from jax.experimental import pallas as pl
from jax.experimental.pallas import tpu as pltpu

_OFF_HALF = 9
_N_SEG = 4
_TH = 128
_RC = 8
_CC = 128


def _loss_kernel(off_ref, flow_ref, out_ref):
    jt = pl.program_id(1)
    w = off_ref.shape[3]

    acc = None
    for r in range(0, _TH, _RC):
        for c in range(0, w, _CC):
            rs = slice(r, r + _RC)
            cs = slice(c, c + _CC)
            ch = [flow_ref[0, k, rs, cs] for k in range(_N_SEG + 1)]
            seg = []
            for j in range(_N_SEG):
                u = ch[j]
                v = ch[j + 1]
                uu = u * u + v * v
                inv = 1.0 / uu
                wj = u * uu
                lo = jnp.minimum(0.0, wj)
                hi = jnp.maximum(0.0, wj)
                hi = jnp.where(uu > 0.0, hi, -1.0)
                seg.append((u, v, inv, lo, hi))
            msum = None
            for i in range(_OFF_HALF):
                x = off_ref[0, i, rs, cs]
                y = off_ref[0, _OFF_HALF + i, rs, cs]
                xx = x * x
                d1sq = xx + y * y
                cy = [ch[k] * y for k in range(_N_SEG + 1)]
                cx = [ch[k] * x for k in range(1, _N_SEG + 1)]
                msq = None
                any_out = None
                for j, (u, v, inv, lo, hi) in enumerate(seg):
                    s = u * (xx + cy[j + 1])
                    out = (s < lo) | (s > hi)
                    t = cx[j] - cy[j]
                    perpsq = t * t * inv
                    dx = x - u
                    dy = y - v
                    d2sq = dx * dx + dy * dy
                    md = jnp.where(out, d2sq, perpsq)
                    msq = md if msq is None else jnp.minimum(msq, md)
                    any_out = out if any_out is None else any_out | out
                msq = jnp.minimum(msq, jnp.where(any_out, d1sq, jnp.inf))
                m = msq * jax.lax.rsqrt(jnp.maximum(msq, 1e-30))
                msum = m if msum is None else msum + m
            acc = msum if acc is None else acc + msum

    s81 = jnp.sum(acc, axis=-1, keepdims=True)
    s11 = jnp.sum(s81, axis=0, keepdims=True)
    part = jnp.broadcast_to(s11, (8, 128))

    @pl.when((pl.program_id(0) == 0) & (jt == 0))
    def _():
        out_ref[...] = jnp.zeros_like(out_ref)

    out_ref[...] += part


def kernel(offset, optical_flow):
    b, c_off, h, w = offset.shape
    of_num = optical_flow.shape[1] // 2
    ht = h // _TH

    out = pl.pallas_call(
        _loss_kernel,
        out_shape=jax.ShapeDtypeStruct((8, 128), jnp.float32),
        grid=(b, ht),
        in_specs=[
            pl.BlockSpec((1, c_off, _TH, w), lambda i, j: (i, 0, j, 0)),
            pl.BlockSpec((1, of_num + 1, _TH, w), lambda i, j: (i, 0, j, 0)),
        ],
        out_specs=pl.BlockSpec((8, 128), lambda i, j: (0, 0)),
        compiler_params=pltpu.CompilerParams(
            dimension_semantics=("arbitrary", "arbitrary"),
        ),
        name="position_loss_val",
    )(offset, optical_flow)

    return out[0, 0] / (_OFF_HALF * h * w)
